# baseline (speedup 1.0000x reference)
import functools

import jax
import jax.numpy as jnp
from jax import lax
from jax.experimental import pallas as pl
from jax.experimental.pallas import tpu as pltpu
from jax.experimental.pallas import tpu_sc as plsc

VOCAB = 1000000
D = 32
B = 16384

G = 8
NC = 2
NS = 16
NW = NC * NS
BPW = B // NW

_mesh = plsc.VectorSubcoreMesh(core_axis_name="c", subcore_axis_name="s")


@functools.partial(
    pl.kernel,
    mesh=_mesh,
    out_type=jax.ShapeDtypeStruct((B, D // 2), jnp.int32),
    compiler_params=pltpu.CompilerParams(needs_layout_passes=False),
    scratch_types=[
        pltpu.VMEM((BPW,), jnp.int32),
        pltpu.VMEM((BPW, D // 2), jnp.int32),
        pltpu.SemaphoreType.DMA,
    ],
)
def _sc_gather(idx_hbm, tbl_hbm, out_hbm, idx_v, rows_v, sem):
    wid = lax.axis_index("s") * NC + lax.axis_index("c")
    base = wid * BPW
    pltpu.sync_copy(idx_hbm.at[pl.ds(base, BPW)], idx_v)

    for b0 in range(0, BPW, 16):
        v = idx_v[pl.ds(b0, 16)]
        for l in range(16):
            idx = v[l]
            pltpu.async_copy(
                tbl_hbm.at[idx >> 3, idx & 7], rows_v.at[b0 + l], sem
            )
    pltpu.make_async_copy(
        out_hbm.at[pl.ds(base, BPW)], rows_v, sem
    ).wait()
    pltpu.sync_copy(rows_v, out_hbm.at[pl.ds(base, BPW)])


def kernel(down_ID, table):
    idx = down_ID.astype(jnp.int32)
    t16 = table.astype(jnp.bfloat16).reshape(VOCAB, D // 2, 2)
    t32 = jax.lax.bitcast_convert_type(t16, jnp.int32)
    tbl = t32.reshape(VOCAB // G, G, D // 2)
    out = _sc_gather(idx, tbl)
    o16 = jax.lax.bitcast_convert_type(out, jnp.bfloat16)
    return o16.reshape(B, D).astype(jnp.float32)


# speedup vs baseline: 4.0986x; 4.0986x over previous
import functools

import jax
import jax.numpy as jnp
from jax import lax
from jax.experimental import pallas as pl
from jax.experimental.pallas import tpu as pltpu
from jax.experimental.pallas import tpu_sc as plsc

VOCAB = 1000000
D = 32
B = 16384

G = 8
NC = 2
NS = 16
NW = NC * NS
BPW = B // NW

_mesh = plsc.VectorSubcoreMesh(core_axis_name="c", subcore_axis_name="s")


@functools.partial(
    pl.kernel,
    mesh=_mesh,
    out_type=jax.ShapeDtypeStruct((B, D), jnp.float32),
    compiler_params=pltpu.CompilerParams(needs_layout_passes=False),
    scratch_types=[
        pltpu.VMEM((BPW,), jnp.int32),
        pltpu.VMEM((BPW, D), jnp.float32),
        pltpu.SemaphoreType.DMA,
    ],
)
def _sc_gather(idx_hbm, tbl_hbm, out_hbm, idx_v, rows_v, sem):
    wid = lax.axis_index("s") * NC + lax.axis_index("c")
    base = wid * BPW
    pltpu.sync_copy(idx_hbm.at[pl.ds(base, BPW)], idx_v)

    for b0 in range(0, BPW, 16):
        v = idx_v[pl.ds(b0, 16)]
        for l in range(16):
            idx = v[l]
            pltpu.async_copy(
                tbl_hbm.at[idx >> 3, idx & 7], rows_v.at[b0 + l], sem
            )
    pltpu.make_async_copy(
        out_hbm.at[pl.ds(base, BPW)], rows_v, sem
    ).wait()
    pltpu.sync_copy(rows_v, out_hbm.at[pl.ds(base, BPW)])


def kernel(down_ID, table):
    idx = down_ID.astype(jnp.int32)
    tbl = table.reshape(VOCAB // G, G, D)
    return _sc_gather(idx, tbl)
